# baseline (speedup 1.0000x reference)
import functools

import jax
import jax.numpy as jnp
from jax import lax
from jax.experimental import pallas as pl
from jax.experimental.pallas import tpu as pltpu
from jax.experimental.pallas import tpu_sc as plsc

B = 64
M = 2048
D = 512
K_STATIC = 50
BLK = 64
L = 16
NGRP = B // L
SCCH = M // 16

W_SCALAR = -0.02830461598932743


def _cos_body(a_ref, b_ref, c0, c1, c2, c3, t0, t1, t2, t3):
    a = a_ref[:]
    b = b_ref[:]
    an = jnp.sqrt(jnp.sum(a * a, axis=1, keepdims=True))
    bn = jnp.sqrt(jnp.sum(b * b, axis=1, keepdims=True))
    sim = lax.dot_general(
        b, a, (((1,), (1,)), ((), ())), preferred_element_type=jnp.float32,
        precision=lax.Precision.HIGHEST,
    )
    cost = sim / jnp.maximum(bn * an.T, 1e-8)

    lo0 = jnp.full((1, B), -1.1, dtype=jnp.float32)
    hi0 = jnp.max(cost, axis=0, keepdims=True) + 1e-6

    def bisect(_, carry):
        lo, hi = carry
        mid = 0.5 * (lo + hi)
        cnt = jnp.sum((cost >= mid).astype(jnp.float32), axis=0,
                      keepdims=True)
        ge = cnt >= K_STATIC
        return jnp.where(ge, mid, lo), jnp.where(ge, hi, mid)

    lo, _ = lax.fori_loop(0, 30, bisect, (lo0, hi0))
    mx = jnp.max(cost, axis=0, keepdims=True)

    for g, (cref, tref) in enumerate(((c0, t0), (c1, t1), (c2, t2), (c3, t3))):
        cref[:] = cost[:, g * L:(g + 1) * L]
        tref[:] = jnp.concatenate(
            [jnp.broadcast_to(lo[:, g * L:(g + 1) * L], (4, L)),
             jnp.broadcast_to(mx[:, g * L:(g + 1) * L], (4, L))], axis=0)


def _sc_weights_body(c0, c1, c2, c3, t0, t1, t2, t3, w0, w1, w2, w3,
                     colbuf, outbuf, scal_buf, part_buf, rbuf, shared):
    cid = lax.axis_index("c")
    sid = lax.axis_index("s")
    h = sid % 16
    zeros_f = jnp.zeros((L,), dtype=jnp.float32)
    rows = pl.ds(h * SCCH, SCCH)

    def work_pair(pair):
        for idx, (cref, tref, _) in enumerate(pair):
            pltpu.sync_copy(cref.at[rows], colbuf.at[idx])
            pltpu.sync_copy(tref, scal_buf.at[idx])
            thv = scal_buf[idx, 0, pl.ds(0, L)]
            mxv = scal_buf[idx, 4, pl.ds(0, L)]

            def ebody(c, acc):
                v = colbuf[idx, c, pl.ds(0, L)]
                e = jnp.where(v >= thv, jnp.exp(v - mxv), zeros_f)
                outbuf[idx, c, pl.ds(0, L)] = e
                return acc + e

            svec = lax.fori_loop(0, SCCH, ebody, zeros_f)
            part_buf[0, pl.ds(0, L)] = svec
            pltpu.sync_copy(part_buf,
                            shared.at[pl.ds(idx * 128 + sid * 8, 8)])

        plsc.subcore_barrier()

        for idx, (_, _, wref) in enumerate(pair):
            pltpu.sync_copy(shared.at[pl.ds(idx * 128, 128)], rbuf)

            def sbody(j, acc):
                return acc + rbuf[j * 8, pl.ds(0, L)]

            total = lax.fori_loop(0, 16, sbody, zeros_f)
            inv = 1.0 / total

            def nbody(c, _):
                outbuf[idx, c, pl.ds(0, L)] = (
                    outbuf[idx, c, pl.ds(0, L)] * inv)
                return 0

            lax.fori_loop(0, SCCH, nbody, 0)
            pltpu.sync_copy(outbuf.at[idx], wref.at[rows])

    @pl.when(cid == 0)
    def _core0():
        work_pair(((c0, t0, w0), (c1, t1, w1)))

    @pl.when(cid == 1)
    def _core1():
        work_pair(((c2, t2, w2), (c3, t3, w3)))


def _matmul_body(w_scalar, mk, num_k, wt0, wt1, wt2, wt3, mf_any, enc_ref,
                 out_ref, buf, sems):
    wt_ref = (wt0, wt1, wt2, wt3)
    rg = pl.program_id(0)
    kstep = pl.program_id(1)
    num_steps = 8 * num_k
    i = rg * num_k + kstep

    mf3 = mf_any.at[0:M * BLK, :].reshape(M, BLK, D)

    def region(step):
        rg_ = step // num_k
        k_ = step % num_k
        return mf3.at[pl.ds(k_ * mk, mk), pl.ds(rg_ * 8, 8), :]

    slot = lax.rem(i, 2)
    nslot = lax.rem(i + 1, 2)

    @pl.when(i == 0)
    def _prime():
        pltpu.make_async_copy(region(0), buf.at[0], sems.at[0]).start()

    @pl.when(i + 1 < num_steps)
    def _prefetch():
        pltpu.make_async_copy(
            region(i + 1), buf.at[nslot], sems.at[nslot]).start()

    pltpu.make_async_copy(region(i), buf.at[slot], sems.at[slot]).wait()

    @pl.when(kstep == 0)
    def _init():
        out_ref[:] = jnp.zeros_like(out_ref)

    wts = jnp.concatenate(
        [w[:] for w in wt_ref], axis=1).astype(jnp.bfloat16)
    t = buf[slot].astype(jnp.bfloat16)
    acc = lax.dot_general(
        wts, t, (((0,), (0,)), ((), ())),
        preferred_element_type=jnp.float32,
    )
    out_ref[:] += acc

    @pl.when(kstep == num_k - 1)
    def _finish():
        out_ref[:] = out_ref[:] * w_scalar + enc_ref[:] * (1.0 - w_scalar)


def kernel(enc_outputs, calculate_memory_context, memory_context, k, memory_fingerprint):
    del k
    a = calculate_memory_context
    b = memory_context

    cmats = jax.ShapeDtypeStruct((M, L), jnp.float32)
    tmats = jax.ShapeDtypeStruct((8, L), jnp.float32)
    c0, c1, c2, c3, t0, t1, t2, t3 = pl.pallas_call(
        _cos_body,
        out_shape=(cmats,) * 4 + (tmats,) * 4,
    )(a, b)

    sc_weights = functools.partial(
        pl.kernel,
        mesh=plsc.VectorSubcoreMesh(core_axis_name="c", subcore_axis_name="s"),
        out_type=(jax.ShapeDtypeStruct((M, L), jnp.float32),) * 4,
        scratch_types=[
            pltpu.VMEM((2, SCCH, L), jnp.float32),
            pltpu.VMEM((2, SCCH, L), jnp.float32),
            pltpu.VMEM((2, 8, L), jnp.float32),
            pltpu.VMEM((8, L), jnp.float32),
            pltpu.VMEM((128, L), jnp.float32),
            pltpu.VMEM_SHARED((256, L), jnp.float32),
        ],
    )(_sc_weights_body)
    w0, w1, w2, w3 = sc_weights(c0, c1, c2, c3, t0, t1, t2, t3)

    mk = 1024
    num_k = M // mk
    wspec = pl.BlockSpec((mk, L), lambda r, k: (k, 0))
    out = pl.pallas_call(
        functools.partial(_matmul_body, W_SCALAR, mk, num_k),
        grid=(BLK // 8, num_k),
        in_specs=[
            wspec, wspec, wspec, wspec,
            pl.BlockSpec(memory_space=pl.ANY),
            pl.BlockSpec((B, 8, D), lambda r, k: (0, r, 0)),
        ],
        out_specs=pl.BlockSpec((B, 8, D), lambda r, k: (0, r, 0)),
        out_shape=jax.ShapeDtypeStruct((B, BLK, D), jnp.float32),
        scratch_shapes=[
            pltpu.VMEM((2, mk, 8, D), jnp.float32),
            pltpu.SemaphoreType.DMA((2,)),
        ],
    )(w0, w1, w2, w3, memory_fingerprint, enc_outputs)

    return out


# speedup vs baseline: 1.0131x; 1.0131x over previous
import functools

import jax
import jax.numpy as jnp
from jax import lax
from jax.experimental import pallas as pl
from jax.experimental.pallas import tpu as pltpu
from jax.experimental.pallas import tpu_sc as plsc

B = 64
M = 2048
D = 512
K_STATIC = 50
BLK = 64
L = 16
NGRP = B // L
SCCH = M // 16

W_SCALAR = -0.02830461598932743


def _cos_body(a_ref, b_ref, c0, c1, c2, c3, t0, t1, t2, t3):
    a = a_ref[:]
    b = b_ref[:]
    an = jnp.sqrt(jnp.sum(a * a, axis=1, keepdims=True))
    bn = jnp.sqrt(jnp.sum(b * b, axis=1, keepdims=True))
    sim = lax.dot_general(
        b, a, (((1,), (1,)), ((), ())), preferred_element_type=jnp.float32,
        precision=lax.Precision.HIGHEST,
    )
    cost = sim / jnp.maximum(bn * an.T, 1e-8)

    lo0 = jnp.full((1, B), -1.1, dtype=jnp.float32)
    hi0 = jnp.max(cost, axis=0, keepdims=True) + 1e-6

    def bisect(_, carry):
        lo, hi = carry
        mid = 0.5 * (lo + hi)
        cnt = jnp.sum((cost >= mid).astype(jnp.float32), axis=0,
                      keepdims=True)
        ge = cnt >= K_STATIC
        return jnp.where(ge, mid, lo), jnp.where(ge, hi, mid)

    lo, _ = lax.fori_loop(0, 30, bisect, (lo0, hi0))
    mx = jnp.max(cost, axis=0, keepdims=True)

    for g, (cref, tref) in enumerate(((c0, t0), (c1, t1), (c2, t2), (c3, t3))):
        cref[:] = cost[:, g * L:(g + 1) * L]
        tref[:] = jnp.concatenate(
            [jnp.broadcast_to(lo[:, g * L:(g + 1) * L], (4, L)),
             jnp.broadcast_to(mx[:, g * L:(g + 1) * L], (4, L))], axis=0)


def _sc_weights_body(c0, c1, c2, c3, t0, t1, t2, t3, w0, w1, w2, w3,
                     colbuf, outbuf, scal_buf, part_buf, rbuf, shared):
    cid = lax.axis_index("c")
    sid = lax.axis_index("s")
    h = sid % 16
    zeros_f = jnp.zeros((L,), dtype=jnp.float32)
    rows = pl.ds(h * SCCH, SCCH)

    def work_pair(pair):
        for idx, (cref, tref, _) in enumerate(pair):
            pltpu.sync_copy(cref.at[rows], colbuf.at[idx])
            pltpu.sync_copy(tref, scal_buf.at[idx])
            thv = scal_buf[idx, 0, pl.ds(0, L)]
            mxv = scal_buf[idx, 4, pl.ds(0, L)]

            def ebody(c4, acc):
                for u in range(4):
                    c = c4 * 4 + u
                    v = colbuf[idx, c, pl.ds(0, L)]
                    e = jnp.where(v >= thv, jnp.exp(v - mxv), zeros_f)
                    outbuf[idx, c, pl.ds(0, L)] = e
                    acc = acc + e
                return acc

            svec = lax.fori_loop(0, SCCH // 4, ebody, zeros_f)
            part_buf[0, pl.ds(0, L)] = svec
            pltpu.sync_copy(part_buf,
                            shared.at[pl.ds(idx * 128 + sid * 8, 8)])

        plsc.subcore_barrier()

        for idx, (_, _, wref) in enumerate(pair):
            pltpu.sync_copy(shared.at[pl.ds(idx * 128, 128)], rbuf)

            def sbody(j, acc):
                return acc + rbuf[j * 8, pl.ds(0, L)]

            total = lax.fori_loop(0, 16, sbody, zeros_f)
            inv = 1.0 / total

            def nbody(c4, _):
                for u in range(4):
                    c = c4 * 4 + u
                    outbuf[idx, c, pl.ds(0, L)] = (
                        outbuf[idx, c, pl.ds(0, L)] * inv)
                return 0

            lax.fori_loop(0, SCCH // 4, nbody, 0)
            pltpu.sync_copy(outbuf.at[idx], wref.at[rows])

    @pl.when(cid == 0)
    def _core0():
        work_pair(((c0, t0, w0), (c1, t1, w1)))

    @pl.when(cid == 1)
    def _core1():
        work_pair(((c2, t2, w2), (c3, t3, w3)))


def _matmul_body(w_scalar, mk, num_k, wt0, wt1, wt2, wt3, mf_any, enc_ref,
                 out_ref, buf, sems):
    wt_ref = (wt0, wt1, wt2, wt3)
    rg = pl.program_id(0)
    kstep = pl.program_id(1)
    num_steps = 8 * num_k
    i = rg * num_k + kstep

    mf3 = mf_any.at[0:M * BLK, :].reshape(M, BLK, D)

    def region(step):
        rg_ = step // num_k
        k_ = step % num_k
        return mf3.at[pl.ds(k_ * mk, mk), pl.ds(rg_ * 8, 8), :]

    slot = lax.rem(i, 2)
    nslot = lax.rem(i + 1, 2)

    @pl.when(i == 0)
    def _prime():
        pltpu.make_async_copy(region(0), buf.at[0], sems.at[0]).start()

    @pl.when(i + 1 < num_steps)
    def _prefetch():
        pltpu.make_async_copy(
            region(i + 1), buf.at[nslot], sems.at[nslot]).start()

    pltpu.make_async_copy(region(i), buf.at[slot], sems.at[slot]).wait()

    @pl.when(kstep == 0)
    def _init():
        out_ref[:] = jnp.zeros_like(out_ref)

    wts = jnp.concatenate(
        [w[:] for w in wt_ref], axis=1).astype(jnp.bfloat16)
    t = buf[slot].astype(jnp.bfloat16)
    acc = lax.dot_general(
        wts, t, (((0,), (0,)), ((), ())),
        preferred_element_type=jnp.float32,
    )
    out_ref[:] += acc

    @pl.when(kstep == num_k - 1)
    def _finish():
        out_ref[:] = out_ref[:] * w_scalar + enc_ref[:] * (1.0 - w_scalar)


def kernel(enc_outputs, calculate_memory_context, memory_context, k, memory_fingerprint):
    del k
    a = calculate_memory_context
    b = memory_context

    cmats = jax.ShapeDtypeStruct((M, L), jnp.float32)
    tmats = jax.ShapeDtypeStruct((8, L), jnp.float32)
    c0, c1, c2, c3, t0, t1, t2, t3 = pl.pallas_call(
        _cos_body,
        out_shape=(cmats,) * 4 + (tmats,) * 4,
    )(a, b)

    sc_weights = functools.partial(
        pl.kernel,
        mesh=plsc.VectorSubcoreMesh(core_axis_name="c", subcore_axis_name="s"),
        out_type=(jax.ShapeDtypeStruct((M, L), jnp.float32),) * 4,
        scratch_types=[
            pltpu.VMEM((2, SCCH, L), jnp.float32),
            pltpu.VMEM((2, SCCH, L), jnp.float32),
            pltpu.VMEM((2, 8, L), jnp.float32),
            pltpu.VMEM((8, L), jnp.float32),
            pltpu.VMEM((128, L), jnp.float32),
            pltpu.VMEM_SHARED((256, L), jnp.float32),
        ],
    )(_sc_weights_body)
    w0, w1, w2, w3 = sc_weights(c0, c1, c2, c3, t0, t1, t2, t3)

    mk = 1024
    num_k = M // mk
    wspec = pl.BlockSpec((mk, L), lambda r, k: (k, 0))
    out = pl.pallas_call(
        functools.partial(_matmul_body, W_SCALAR, mk, num_k),
        grid=(BLK // 8, num_k),
        in_specs=[
            wspec, wspec, wspec, wspec,
            pl.BlockSpec(memory_space=pl.ANY),
            pl.BlockSpec((B, 8, D), lambda r, k: (0, r, 0)),
        ],
        out_specs=pl.BlockSpec((B, 8, D), lambda r, k: (0, r, 0)),
        out_shape=jax.ShapeDtypeStruct((B, BLK, D), jnp.float32),
        scratch_shapes=[
            pltpu.VMEM((2, mk, 8, D), jnp.float32),
            pltpu.SemaphoreType.DMA((2,)),
        ],
    )(w0, w1, w2, w3, memory_fingerprint, enc_outputs)

    return out


# speedup vs baseline: 1.0744x; 1.0605x over previous
import functools

import jax
import jax.numpy as jnp
from jax import lax
from jax.experimental import pallas as pl
from jax.experimental.pallas import tpu as pltpu
from jax.experimental.pallas import tpu_sc as plsc

B = 64
M = 2048
D = 512
K_STATIC = 50
BLK = 64
L = 16
NGRP = B // L
SCCH = M // 16

W_SCALAR = -0.02830461598932743


def _cos_body(a_ref, b_ref, c0, c1, c2, c3, t0, t1, t2, t3):
    a = a_ref[:]
    b = b_ref[:]
    an = jnp.sqrt(jnp.sum(a * a, axis=1, keepdims=True))
    bn = jnp.sqrt(jnp.sum(b * b, axis=1, keepdims=True))
    sim = lax.dot_general(
        b, a, (((1,), (1,)), ((), ())), preferred_element_type=jnp.float32,
        precision=lax.Precision.HIGHEST,
    )
    cost = sim / jnp.maximum(bn * an.T, 1e-8)

    lo0 = jnp.full((1, B), -1.1, dtype=jnp.float32)
    hi0 = jnp.max(cost, axis=0, keepdims=True) + 1e-6

    def bisect(_, carry):
        lo, hi = carry
        mid = 0.5 * (lo + hi)
        cnt = jnp.sum((cost >= mid).astype(jnp.float32), axis=0,
                      keepdims=True)
        ge = cnt >= K_STATIC
        return jnp.where(ge, mid, lo), jnp.where(ge, hi, mid)

    lo, _ = lax.fori_loop(0, 30, bisect, (lo0, hi0))
    mx = jnp.max(cost, axis=0, keepdims=True)

    for g, (cref, tref) in enumerate(((c0, t0), (c1, t1), (c2, t2), (c3, t3))):
        cref[:] = cost[:, g * L:(g + 1) * L]
        tref[:] = jnp.concatenate(
            [jnp.broadcast_to(lo[:, g * L:(g + 1) * L], (4, L)),
             jnp.broadcast_to(mx[:, g * L:(g + 1) * L], (4, L))], axis=0)


def _sc_weights_body(c0, c1, c2, c3, t0, t1, t2, t3, w0, w1, w2, w3,
                     colbuf, outbuf, scal_buf, part_buf, rbuf, shared):
    cid = lax.axis_index("c")
    sid = lax.axis_index("s")
    h = sid % 16
    zeros_f = jnp.zeros((L,), dtype=jnp.float32)
    rows = pl.ds(h * SCCH, SCCH)

    def work_pair(pair):
        for idx, (cref, tref, _) in enumerate(pair):
            pltpu.sync_copy(cref.at[rows], colbuf.at[idx])
            pltpu.sync_copy(tref, scal_buf.at[idx])
            thv = scal_buf[idx, 0, pl.ds(0, L)]
            mxv = scal_buf[idx, 4, pl.ds(0, L)]

            def ebody(c4, acc):
                for u in range(4):
                    c = c4 * 4 + u
                    v = colbuf[idx, c, pl.ds(0, L)]
                    e = jnp.where(v >= thv, jnp.exp(v - mxv), zeros_f)
                    outbuf[idx, c, pl.ds(0, L)] = e
                    acc = acc + e
                return acc

            svec = lax.fori_loop(0, SCCH // 4, ebody, zeros_f)
            part_buf[0, pl.ds(0, L)] = svec
            pltpu.sync_copy(part_buf,
                            shared.at[pl.ds(idx * 128 + sid * 8, 8)])

        plsc.subcore_barrier()

        for idx, (_, _, wref) in enumerate(pair):
            pltpu.sync_copy(shared.at[pl.ds(idx * 128, 128)], rbuf)

            def sbody(j, acc):
                return acc + rbuf[j * 8, pl.ds(0, L)]

            total = lax.fori_loop(0, 16, sbody, zeros_f)
            inv = 1.0 / total

            def nbody(c4, _):
                for u in range(4):
                    c = c4 * 4 + u
                    outbuf[idx, c, pl.ds(0, L)] = (
                        outbuf[idx, c, pl.ds(0, L)] * inv)
                return 0

            lax.fori_loop(0, SCCH // 4, nbody, 0)
            pltpu.sync_copy(outbuf.at[idx], wref.at[rows])

    @pl.when(cid == 0)
    def _core0():
        work_pair(((c0, t0, w0), (c1, t1, w1)))

    @pl.when(cid == 1)
    def _core1():
        work_pair(((c2, t2, w2), (c3, t3, w3)))


def _matmul_body(w_scalar, mk, num_k, wt0, wt1, wt2, wt3, mf_any, enc_ref,
                 out_ref, wsc, buf, sems):
    wt_ref = (wt0, wt1, wt2, wt3)
    rg = pl.program_id(0)
    kstep = pl.program_id(1)
    num_steps = 8 * num_k
    i = rg * num_k + kstep

    mf3 = mf_any.at[0:M * BLK, :].reshape(M, BLK, D)

    def region(step):
        rg_ = step // num_k
        k_ = step % num_k
        return mf3.at[pl.ds(k_ * mk, mk), pl.ds(rg_ * 8, 8), :]

    slot = lax.rem(i, 2)
    nslot = lax.rem(i + 1, 2)

    @pl.when(i == 0)
    def _prime():
        pltpu.make_async_copy(region(0), buf.at[0], sems.at[0]).start()

    @pl.when(i + 1 < num_steps)
    def _prefetch():
        pltpu.make_async_copy(
            region(i + 1), buf.at[nslot], sems.at[nslot]).start()

    @pl.when(i == 0)
    def _assemble():
        wsc[:] = jnp.concatenate(
            [w[:] for w in wt_ref], axis=1).astype(jnp.bfloat16)

    pltpu.make_async_copy(region(i), buf.at[slot], sems.at[slot]).wait()

    @pl.when(kstep == 0)
    def _init():
        out_ref[:] = jnp.zeros_like(out_ref)

    wts = wsc[pl.ds(kstep * mk, mk), :]
    t = buf[slot].astype(jnp.bfloat16)
    acc = lax.dot_general(
        wts, t, (((0,), (0,)), ((), ())),
        preferred_element_type=jnp.float32,
    )
    out_ref[:] += acc

    @pl.when(kstep == num_k - 1)
    def _finish():
        out_ref[:] = out_ref[:] * w_scalar + enc_ref[:] * (1.0 - w_scalar)


def kernel(enc_outputs, calculate_memory_context, memory_context, k, memory_fingerprint):
    del k
    a = calculate_memory_context
    b = memory_context

    cmats = jax.ShapeDtypeStruct((M, L), jnp.float32)
    tmats = jax.ShapeDtypeStruct((8, L), jnp.float32)
    c0, c1, c2, c3, t0, t1, t2, t3 = pl.pallas_call(
        _cos_body,
        out_shape=(cmats,) * 4 + (tmats,) * 4,
    )(a, b)

    sc_weights = functools.partial(
        pl.kernel,
        mesh=plsc.VectorSubcoreMesh(core_axis_name="c", subcore_axis_name="s"),
        out_type=(jax.ShapeDtypeStruct((M, L), jnp.float32),) * 4,
        scratch_types=[
            pltpu.VMEM((2, SCCH, L), jnp.float32),
            pltpu.VMEM((2, SCCH, L), jnp.float32),
            pltpu.VMEM((2, 8, L), jnp.float32),
            pltpu.VMEM((8, L), jnp.float32),
            pltpu.VMEM((128, L), jnp.float32),
            pltpu.VMEM_SHARED((256, L), jnp.float32),
        ],
    )(_sc_weights_body)
    w0, w1, w2, w3 = sc_weights(c0, c1, c2, c3, t0, t1, t2, t3)

    mk = 1024
    num_k = M // mk
    wspec = pl.BlockSpec((M, L), lambda r, k: (0, 0))
    out = pl.pallas_call(
        functools.partial(_matmul_body, W_SCALAR, mk, num_k),
        grid=(BLK // 8, num_k),
        in_specs=[
            wspec, wspec, wspec, wspec,
            pl.BlockSpec(memory_space=pl.ANY),
            pl.BlockSpec((B, 8, D), lambda r, k: (0, r, 0)),
        ],
        out_specs=pl.BlockSpec((B, 8, D), lambda r, k: (0, r, 0)),
        out_shape=jax.ShapeDtypeStruct((B, BLK, D), jnp.float32),
        scratch_shapes=[
            pltpu.VMEM((M, B), jnp.bfloat16),
            pltpu.VMEM((2, mk, 8, D), jnp.float32),
            pltpu.SemaphoreType.DMA((2,)),
        ],
    )(w0, w1, w2, w3, memory_fingerprint, enc_outputs)

    return out


# speedup vs baseline: 1.0981x; 1.0220x over previous
import functools

import jax
import jax.numpy as jnp
from jax import lax
from jax.experimental import pallas as pl
from jax.experimental.pallas import tpu as pltpu
from jax.experimental.pallas import tpu_sc as plsc

B = 64
M = 2048
D = 512
K_STATIC = 50
BLK = 64
L = 16
NGRP = B // L
SCCH = M // 16

W_SCALAR = -0.02830461598932743


def _cos_body(a_ref, b_ref, c0, c1, c2, c3, t0, t1, t2, t3):
    a = a_ref[:]
    b = b_ref[:]
    an = jnp.sqrt(jnp.sum(a * a, axis=1, keepdims=True))
    bn = jnp.sqrt(jnp.sum(b * b, axis=1, keepdims=True))
    sim = lax.dot_general(
        b, a, (((1,), (1,)), ((), ())), preferred_element_type=jnp.float32,
        precision=lax.Precision.HIGHEST,
    )
    cost = sim / jnp.maximum(bn * an.T, 1e-8)

    lo0 = jnp.full((1, B), -1.1, dtype=jnp.float32)
    hi0 = jnp.max(cost, axis=0, keepdims=True) + 1e-6

    ones_row = jnp.ones((8, M), dtype=jnp.bfloat16)

    def bisect(_, carry):
        lo, hi = carry
        mid = 0.5 * (lo + hi)
        mask = (cost >= mid).astype(jnp.bfloat16)
        cnt = lax.dot_general(
            ones_row, mask, (((1,), (0,)), ((), ())),
            preferred_element_type=jnp.float32,
        )[0:1]
        ge = cnt >= K_STATIC
        return jnp.where(ge, mid, lo), jnp.where(ge, hi, mid)

    lo, _ = lax.fori_loop(0, 30, bisect, (lo0, hi0))
    mx = jnp.max(cost, axis=0, keepdims=True)

    for g, (cref, tref) in enumerate(((c0, t0), (c1, t1), (c2, t2), (c3, t3))):
        cref[:] = cost[:, g * L:(g + 1) * L]
        tref[:] = jnp.concatenate(
            [jnp.broadcast_to(lo[:, g * L:(g + 1) * L], (4, L)),
             jnp.broadcast_to(mx[:, g * L:(g + 1) * L], (4, L))], axis=0)


def _sc_weights_body(c0, c1, c2, c3, t0, t1, t2, t3, w0, w1, w2, w3,
                     colbuf, outbuf, scal_buf, part_buf, rbuf, shared):
    cid = lax.axis_index("c")
    sid = lax.axis_index("s")
    h = sid % 16
    zeros_f = jnp.zeros((L,), dtype=jnp.float32)
    rows = pl.ds(h * SCCH, SCCH)

    def work_pair(pair):
        for idx, (cref, tref, _) in enumerate(pair):
            pltpu.sync_copy(cref.at[rows], colbuf.at[idx])
            pltpu.sync_copy(tref, scal_buf.at[idx])
            thv = scal_buf[idx, 0, pl.ds(0, L)]
            mxv = scal_buf[idx, 4, pl.ds(0, L)]

            def ebody(c4, acc):
                for u in range(4):
                    c = c4 * 4 + u
                    v = colbuf[idx, c, pl.ds(0, L)]
                    e = jnp.where(v >= thv, jnp.exp(v - mxv), zeros_f)
                    outbuf[idx, c, pl.ds(0, L)] = e
                    acc = acc + e
                return acc

            svec = lax.fori_loop(0, SCCH // 4, ebody, zeros_f)
            part_buf[0, pl.ds(0, L)] = svec
            pltpu.sync_copy(part_buf,
                            shared.at[pl.ds(idx * 128 + sid * 8, 8)])

        plsc.subcore_barrier()

        for idx, (_, _, wref) in enumerate(pair):
            pltpu.sync_copy(shared.at[pl.ds(idx * 128, 128)], rbuf)

            def sbody(j, acc):
                return acc + rbuf[j * 8, pl.ds(0, L)]

            total = lax.fori_loop(0, 16, sbody, zeros_f)
            inv = 1.0 / total

            def nbody(c4, _):
                for u in range(4):
                    c = c4 * 4 + u
                    outbuf[idx, c, pl.ds(0, L)] = (
                        outbuf[idx, c, pl.ds(0, L)] * inv)
                return 0

            lax.fori_loop(0, SCCH // 4, nbody, 0)
            pltpu.sync_copy(outbuf.at[idx], wref.at[rows])

    @pl.when(cid == 0)
    def _core0():
        work_pair(((c0, t0, w0), (c1, t1, w1)))

    @pl.when(cid == 1)
    def _core1():
        work_pair(((c2, t2, w2), (c3, t3, w3)))


def _matmul_body(w_scalar, mk, num_k, wt0, wt1, wt2, wt3, mf_any, enc_ref,
                 out_ref, wsc, buf, sems):
    wt_ref = (wt0, wt1, wt2, wt3)
    rg = pl.program_id(0)
    kstep = pl.program_id(1)
    num_steps = 8 * num_k
    i = rg * num_k + kstep

    mf3 = mf_any.at[0:M * BLK, :].reshape(M, BLK, D)

    def region(step):
        rg_ = step // num_k
        k_ = step % num_k
        return mf3.at[pl.ds(k_ * mk, mk), pl.ds(rg_ * 8, 8), :]

    slot = lax.rem(i, 2)
    nslot = lax.rem(i + 1, 2)

    @pl.when(i == 0)
    def _prime():
        pltpu.make_async_copy(region(0), buf.at[0], sems.at[0]).start()

    @pl.when(i + 1 < num_steps)
    def _prefetch():
        pltpu.make_async_copy(
            region(i + 1), buf.at[nslot], sems.at[nslot]).start()

    @pl.when(i == 0)
    def _assemble():
        wsc[:] = jnp.concatenate(
            [w[:] for w in wt_ref], axis=1).astype(jnp.bfloat16)

    pltpu.make_async_copy(region(i), buf.at[slot], sems.at[slot]).wait()

    @pl.when(kstep == 0)
    def _init():
        out_ref[:] = jnp.zeros_like(out_ref)

    wts = wsc[pl.ds(kstep * mk, mk), :]
    t = buf[slot].astype(jnp.bfloat16)
    acc = lax.dot_general(
        wts, t, (((0,), (0,)), ((), ())),
        preferred_element_type=jnp.float32,
    )
    out_ref[:] += acc

    @pl.when(kstep == num_k - 1)
    def _finish():
        out_ref[:] = out_ref[:] * w_scalar + enc_ref[:] * (1.0 - w_scalar)


def kernel(enc_outputs, calculate_memory_context, memory_context, k, memory_fingerprint):
    del k
    a = calculate_memory_context
    b = memory_context

    cmats = jax.ShapeDtypeStruct((M, L), jnp.float32)
    tmats = jax.ShapeDtypeStruct((8, L), jnp.float32)
    c0, c1, c2, c3, t0, t1, t2, t3 = pl.pallas_call(
        _cos_body,
        out_shape=(cmats,) * 4 + (tmats,) * 4,
    )(a, b)

    sc_weights = functools.partial(
        pl.kernel,
        mesh=plsc.VectorSubcoreMesh(core_axis_name="c", subcore_axis_name="s"),
        out_type=(jax.ShapeDtypeStruct((M, L), jnp.float32),) * 4,
        scratch_types=[
            pltpu.VMEM((2, SCCH, L), jnp.float32),
            pltpu.VMEM((2, SCCH, L), jnp.float32),
            pltpu.VMEM((2, 8, L), jnp.float32),
            pltpu.VMEM((8, L), jnp.float32),
            pltpu.VMEM((128, L), jnp.float32),
            pltpu.VMEM_SHARED((256, L), jnp.float32),
        ],
    )(_sc_weights_body)
    w0, w1, w2, w3 = sc_weights(c0, c1, c2, c3, t0, t1, t2, t3)

    mk = 1024
    num_k = M // mk
    wspec = pl.BlockSpec((M, L), lambda r, k: (0, 0))
    out = pl.pallas_call(
        functools.partial(_matmul_body, W_SCALAR, mk, num_k),
        grid=(BLK // 8, num_k),
        in_specs=[
            wspec, wspec, wspec, wspec,
            pl.BlockSpec(memory_space=pl.ANY),
            pl.BlockSpec((B, 8, D), lambda r, k: (0, r, 0)),
        ],
        out_specs=pl.BlockSpec((B, 8, D), lambda r, k: (0, r, 0)),
        out_shape=jax.ShapeDtypeStruct((B, BLK, D), jnp.float32),
        scratch_shapes=[
            pltpu.VMEM((M, B), jnp.bfloat16),
            pltpu.VMEM((2, mk, 8, D), jnp.float32),
            pltpu.SemaphoreType.DMA((2,)),
        ],
    )(w0, w1, w2, w3, memory_fingerprint, enc_outputs)

    return out


# speedup vs baseline: 1.1072x; 1.0083x over previous
import functools

import jax
import jax.numpy as jnp
from jax import lax
from jax.experimental import pallas as pl
from jax.experimental.pallas import tpu as pltpu
from jax.experimental.pallas import tpu_sc as plsc

B = 64
M = 2048
D = 512
K_STATIC = 50
BLK = 64
L = 16
NGRP = B // L
SCCH = M // 16

W_SCALAR = -0.02830461598932743


def _cos_body(a_ref, b_ref, c0, c1, c2, c3, t0, t1, t2, t3):
    a = a_ref[:]
    b = b_ref[:]
    an = jnp.sqrt(jnp.sum(a * a, axis=1, keepdims=True))
    bn = jnp.sqrt(jnp.sum(b * b, axis=1, keepdims=True))
    sim = lax.dot_general(
        b, a, (((1,), (1,)), ((), ())), preferred_element_type=jnp.float32,
        precision=lax.Precision.HIGHEST,
    )
    cost = sim / jnp.maximum(bn * an.T, 1e-8)

    lo0 = jnp.full((1, B), -1.1, dtype=jnp.float32)
    hi0 = jnp.max(cost, axis=0, keepdims=True) + 1e-6

    ones_row = jnp.ones((8, M), dtype=jnp.bfloat16)

    def bisect(_, carry):
        lo, hi = carry
        mid = 0.5 * (lo + hi)
        mask = (cost >= mid).astype(jnp.bfloat16)
        cnt = lax.dot_general(
            ones_row, mask, (((1,), (0,)), ((), ())),
            preferred_element_type=jnp.float32,
        )[0:1]
        ge = cnt >= K_STATIC
        return jnp.where(ge, mid, lo), jnp.where(ge, hi, mid)

    lo, _ = lax.fori_loop(0, 30, bisect, (lo0, hi0))
    mx = jnp.max(cost, axis=0, keepdims=True)

    for g, (cref, tref) in enumerate(((c0, t0), (c1, t1), (c2, t2), (c3, t3))):
        cref[:] = cost[:, g * L:(g + 1) * L]
        tref[:] = jnp.concatenate(
            [jnp.broadcast_to(lo[:, g * L:(g + 1) * L], (4, L)),
             jnp.broadcast_to(mx[:, g * L:(g + 1) * L], (4, L))], axis=0)


def _sc_weights_body(c0, c1, c2, c3, t0, t1, t2, t3,
                     w0, w1, w2, w3, p0, p1, p2, p3,
                     colbuf, outbuf, scal_buf, part_buf):
    cid = lax.axis_index("c")
    sid = lax.axis_index("s")
    zeros_f = jnp.zeros((L,), dtype=jnp.float32)
    rows = pl.ds(sid * SCCH, SCCH)

    def work_pair(pair):
        for idx, (cref, tref, wref, pref) in enumerate(pair):
            pltpu.sync_copy(cref.at[rows], colbuf.at[idx])
            pltpu.sync_copy(tref, scal_buf.at[idx])
            thv = scal_buf[idx, 0, pl.ds(0, L)]
            mxv = scal_buf[idx, 4, pl.ds(0, L)]

            def ebody(c4, acc):
                for u in range(4):
                    c = c4 * 4 + u
                    v = colbuf[idx, c, pl.ds(0, L)]
                    e = jnp.where(v >= thv, jnp.exp(v - mxv), zeros_f)
                    outbuf[idx, c, pl.ds(0, L)] = e
                    acc = acc + e
                return acc

            svec = lax.fori_loop(0, SCCH // 4, ebody, zeros_f)
            for j in range(8):
                part_buf[j, pl.ds(0, L)] = svec
            pltpu.sync_copy(outbuf.at[idx], wref.at[rows])
            pltpu.sync_copy(part_buf, pref.at[pl.ds(sid * 8, 8)])

    @pl.when(cid == 0)
    def _core0():
        work_pair(((c0, t0, w0, p0), (c1, t1, w1, p1)))

    @pl.when(cid == 1)
    def _core1():
        work_pair(((c2, t2, w2, p2), (c3, t3, w3, p3)))


def _matmul_body(w_scalar, mk, num_k, wt0, wt1, wt2, wt3,
                 p0, p1, p2, p3, mf_any, enc_ref,
                 out_ref, wsc, buf, sems):
    wt_ref = (wt0, wt1, wt2, wt3)
    p_ref = (p0, p1, p2, p3)
    rg = pl.program_id(0)
    kstep = pl.program_id(1)
    num_steps = 8 * num_k
    i = rg * num_k + kstep

    mf3 = mf_any.at[0:M * BLK, :].reshape(M, BLK, D)

    def region(step):
        rg_ = step // num_k
        k_ = step % num_k
        return mf3.at[pl.ds(k_ * mk, mk), pl.ds(rg_ * 8, 8), :]

    slot = lax.rem(i, 2)
    nslot = lax.rem(i + 1, 2)

    @pl.when(i == 0)
    def _prime():
        pltpu.make_async_copy(region(0), buf.at[0], sems.at[0]).start()

    @pl.when(i + 1 < num_steps)
    def _prefetch():
        pltpu.make_async_copy(
            region(i + 1), buf.at[nslot], sems.at[nslot]).start()

    @pl.when(i == 0)
    def _assemble():
        tot = jnp.concatenate(
            [jnp.sum(p[:], axis=0, keepdims=True) for p in p_ref],
            axis=1) * 0.125
        e_all = jnp.concatenate([w[:] for w in wt_ref], axis=1)
        wsc[:] = (e_all * (1.0 / tot)).astype(jnp.bfloat16)

    pltpu.make_async_copy(region(i), buf.at[slot], sems.at[slot]).wait()

    @pl.when(kstep == 0)
    def _init():
        out_ref[:] = jnp.zeros_like(out_ref)

    wts = wsc[pl.ds(kstep * mk, mk), :]
    t = buf[slot].astype(jnp.bfloat16)
    acc = lax.dot_general(
        wts, t, (((0,), (0,)), ((), ())),
        preferred_element_type=jnp.float32,
    )
    out_ref[:] += acc

    @pl.when(kstep == num_k - 1)
    def _finish():
        out_ref[:] = out_ref[:] * w_scalar + enc_ref[:] * (1.0 - w_scalar)


def kernel(enc_outputs, calculate_memory_context, memory_context, k, memory_fingerprint):
    del k
    a = calculate_memory_context
    b = memory_context

    cmats = jax.ShapeDtypeStruct((M, L), jnp.float32)
    tmats = jax.ShapeDtypeStruct((8, L), jnp.float32)
    c0, c1, c2, c3, t0, t1, t2, t3 = pl.pallas_call(
        _cos_body,
        out_shape=(cmats,) * 4 + (tmats,) * 4,
    )(a, b)

    sc_weights = functools.partial(
        pl.kernel,
        mesh=plsc.VectorSubcoreMesh(core_axis_name="c", subcore_axis_name="s"),
        out_type=(jax.ShapeDtypeStruct((M, L), jnp.float32),) * 4
        + (jax.ShapeDtypeStruct((128, L), jnp.float32),) * 4,
        scratch_types=[
            pltpu.VMEM((2, SCCH, L), jnp.float32),
            pltpu.VMEM((2, SCCH, L), jnp.float32),
            pltpu.VMEM((2, 8, L), jnp.float32),
            pltpu.VMEM((8, L), jnp.float32),
        ],
    )(_sc_weights_body)
    w0, w1, w2, w3, p0, p1, p2, p3 = sc_weights(
        c0, c1, c2, c3, t0, t1, t2, t3)

    mk = 1024
    num_k = M // mk
    wspec = pl.BlockSpec((M, L), lambda r, k: (0, 0))
    out = pl.pallas_call(
        functools.partial(_matmul_body, W_SCALAR, mk, num_k),
        grid=(BLK // 8, num_k),
        in_specs=[
            wspec, wspec, wspec, wspec,
            pl.BlockSpec((128, L), lambda r, k: (0, 0)),
            pl.BlockSpec((128, L), lambda r, k: (0, 0)),
            pl.BlockSpec((128, L), lambda r, k: (0, 0)),
            pl.BlockSpec((128, L), lambda r, k: (0, 0)),
            pl.BlockSpec(memory_space=pl.ANY),
            pl.BlockSpec((B, 8, D), lambda r, k: (0, r, 0)),
        ],
        out_specs=pl.BlockSpec((B, 8, D), lambda r, k: (0, r, 0)),
        out_shape=jax.ShapeDtypeStruct((B, BLK, D), jnp.float32),
        scratch_shapes=[
            pltpu.VMEM((M, B), jnp.bfloat16),
            pltpu.VMEM((2, mk, 8, D), jnp.float32),
            pltpu.SemaphoreType.DMA((2,)),
        ],
    )(w0, w1, w2, w3, p0, p1, p2, p3, memory_fingerprint, enc_outputs)

    return out
